# Initial kernel scaffold; baseline (speedup 1.0000x reference)
import jax
import jax.numpy as jnp
from jax.experimental import pallas as pl


def kernel(x):
    raise NotImplementedError("write your pallas kernel here")


# speedup vs baseline: 19.4318x; 19.4318x over previous
import jax
import jax.numpy as jnp
from jax import lax
from jax.experimental import pallas as pl
from jax.experimental.pallas import tpu as pltpu

_K = 81
_BR = 256


def _wta_body(x_ref, o_ref):
    xb = x_ref[...]
    b = lax.bitcast_convert_type(xb, jnp.int32)
    skey = b ^ ((b >> 31) & jnp.int32(0x7FFFFFFF))
    rows = xb.shape[0]
    t = jnp.full((rows, 1), jnp.int32(-2147483648))
    steps = [jnp.int32(-2147483648)] + [jnp.int32(1 << s) for s in range(30, -1, -1)]
    for step in steps:
        cand = t + step
        cnt = jnp.sum((skey >= cand).astype(jnp.int32), axis=1, keepdims=True)
        t = jnp.where(cnt >= _K, cand, t)
    o_ref[...] = jnp.where(skey >= t, xb, jnp.float32(0.0))


def kernel(x):
    B, S, N = x.shape
    rows = B * S
    xf = x.reshape(rows, N)
    br = _BR if rows % _BR == 0 else rows
    out = pl.pallas_call(
        _wta_body,
        grid=(rows // br,),
        in_specs=[pl.BlockSpec((br, N), lambda i: (i, 0))],
        out_specs=pl.BlockSpec((br, N), lambda i: (i, 0)),
        out_shape=jax.ShapeDtypeStruct((rows, N), jnp.float32),
        compiler_params=pltpu.CompilerParams(
            dimension_semantics=("parallel",),
        ),
    )(xf)
    return out.reshape(B, S, N)
